# baseline (speedup 1.0000x reference)
import functools

import jax
import jax.numpy as jnp
from jax import lax
from jax.experimental import pallas as pl
from jax.experimental.pallas import tpu as pltpu
from jax.experimental.pallas import tpu_sc as plsc

_CH = 64
_NBUF = 3
_LOOK = 2


def _sc_masked_copy(x_hbm, z_hbm, o_hbm, bufs, in_sems, out_sems):
    nb, t, w = x_hbm.shape
    nchunk = t // _CH
    wid = lax.axis_index("s") * 2 + lax.axis_index("c")

    @pl.when(wid == 0)
    def _zero_batch0():
        pltpu.make_async_copy(z_hbm, bufs.at[0], in_sems.at[0]).start()
        pltpu.make_async_copy(z_hbm, bufs.at[0], in_sems.at[0]).wait()
        for c0 in range(0, nchunk, _NBUF):
            for c in range(c0, min(c0 + _NBUF, nchunk)):
                pltpu.make_async_copy(
                    bufs.at[0],
                    o_hbm.at[0, pl.ds(c * _CH, _CH), :],
                    out_sems.at[c - c0],
                ).start()
            for c in range(c0, min(c0 + _NBUF, nchunk)):
                pltpu.make_async_copy(
                    bufs.at[0],
                    o_hbm.at[0, pl.ds(c * _CH, _CH), :],
                    out_sems.at[c - c0],
                ).wait()

    @pl.when(wid > 0)
    def _copy_batch():
        def src(c):
            return x_hbm.at[wid, pl.ds(c * _CH, _CH), :]

        def dst(c):
            return o_hbm.at[wid, pl.ds(c * _CH, _CH), :]

        for c in range(_LOOK):
            pltpu.make_async_copy(src(c), bufs.at[c % _NBUF], in_sems.at[c % _NBUF]).start()
        for c in range(nchunk):
            b = c % _NBUF
            pltpu.make_async_copy(src(c), bufs.at[b], in_sems.at[b]).wait()
            pltpu.make_async_copy(bufs.at[b], dst(c), out_sems.at[b]).start()
            cn = c + _LOOK
            if cn < nchunk:
                bn = cn % _NBUF
                if cn >= _NBUF:
                    pltpu.make_async_copy(
                        bufs.at[bn], dst(cn - _NBUF), out_sems.at[bn]
                    ).wait()
                pltpu.make_async_copy(src(cn), bufs.at[bn], in_sems.at[bn]).start()
        for c in range(max(0, nchunk - _NBUF), nchunk):
            b = c % _NBUF
            pltpu.make_async_copy(bufs.at[b], dst(c), out_sems.at[b]).wait()


def kernel(time_images_season_list):
    x = time_images_season_list
    _, b, t, c, n = x.shape
    wdt = c * n
    x2 = x.reshape(b, t, wdt)
    z = jnp.zeros((_CH, wdt), x.dtype)
    mesh = plsc.VectorSubcoreMesh(core_axis_name="c", subcore_axis_name="s")
    run = pl.kernel(
        _sc_masked_copy,
        mesh=mesh,
        out_type=jax.ShapeDtypeStruct((b, t, wdt), x.dtype),
        scratch_types=[
            pltpu.VMEM((_NBUF, _CH, wdt), x.dtype),
            pltpu.SemaphoreType.DMA((_NBUF,)),
            pltpu.SemaphoreType.DMA((_NBUF,)),
        ],
    )
    out = run(x2, z)
    return out.reshape(b, t, c, n)


# speedup vs baseline: 1.0014x; 1.0014x over previous
import jax
import jax.numpy as jnp
from jax import lax
from jax.experimental import pallas as pl
from jax.experimental.pallas import tpu as pltpu
from jax.experimental.pallas import tpu_sc as plsc

_CH = 64
_NBUF = 3
_LOOK = 2


def _sc_masked_copy(x_hbm, z_hbm, o_hbm, bufs, in_sems, out_sems):
    nb, t, w = x_hbm.shape
    nchunk = t // _CH
    wid = lax.axis_index("s") * 2 + lax.axis_index("c")

    @pl.when(wid == 0)
    def _zero_batch0():
        pltpu.make_async_copy(z_hbm, bufs.at[0], in_sems.at[0]).start()
        pltpu.make_async_copy(z_hbm, bufs.at[0], in_sems.at[0]).wait()
        for c0 in range(0, nchunk, _NBUF):
            for c in range(c0, min(c0 + _NBUF, nchunk)):
                pltpu.make_async_copy(
                    bufs.at[0],
                    o_hbm.at[0, pl.ds(c * _CH, _CH), :],
                    out_sems.at[c - c0],
                ).start()
            for c in range(c0, min(c0 + _NBUF, nchunk)):
                pltpu.make_async_copy(
                    bufs.at[0],
                    o_hbm.at[0, pl.ds(c * _CH, _CH), :],
                    out_sems.at[c - c0],
                ).wait()

    @pl.when(wid > 0)
    def _copy_batch():
        def src(c):
            return x_hbm.at[wid, pl.ds(c * _CH, _CH), :]

        def dst(c):
            return o_hbm.at[wid, pl.ds(c * _CH, _CH), :]

        for c in range(_LOOK):
            pltpu.make_async_copy(src(c), bufs.at[c % _NBUF], in_sems.at[c % _NBUF]).start()
        for c in range(nchunk):
            b = c % _NBUF
            pltpu.make_async_copy(src(c), bufs.at[b], in_sems.at[b]).wait()
            pltpu.make_async_copy(bufs.at[b], dst(c), out_sems.at[b]).start()
            cn = c + _LOOK
            if cn < nchunk:
                bn = cn % _NBUF
                if cn >= _NBUF:
                    pltpu.make_async_copy(
                        bufs.at[bn], dst(cn - _NBUF), out_sems.at[bn]
                    ).wait()
                pltpu.make_async_copy(src(cn), bufs.at[bn], in_sems.at[bn]).start()
        for c in range(max(0, nchunk - _NBUF), nchunk):
            b = c % _NBUF
            pltpu.make_async_copy(bufs.at[b], dst(c), out_sems.at[b]).wait()


def kernel(time_images_season_list):
    x = time_images_season_list
    _, b, t, c, n = x.shape
    wdt = c * n
    x2 = x.reshape(b, t, wdt)
    z = jnp.zeros((_CH, wdt), x.dtype)
    mesh = plsc.VectorSubcoreMesh(core_axis_name="c", subcore_axis_name="s")
    run = pl.kernel(
        _sc_masked_copy,
        mesh=mesh,
        out_type=jax.ShapeDtypeStruct((b, t, wdt), x.dtype),
        scratch_types=[
            pltpu.VMEM((_NBUF, _CH, wdt), x.dtype),
            pltpu.SemaphoreType.DMA((_NBUF,)),
            pltpu.SemaphoreType.DMA((_NBUF,)),
        ],
    )
    out = run(x2, z)
    return out.reshape(b, t, c, n)
